# Initial kernel scaffold; baseline (speedup 1.0000x reference)
import jax
import jax.numpy as jnp
from jax.experimental import pallas as pl


def kernel(x, edge_index, W_enc, b_enc, W1, b1, W_gcn, b_gcn, W_out, b_out, W_v, b_v):
    raise NotImplementedError("write your pallas kernel here")


# speedup vs baseline: 7.3994x; 7.3994x over previous
import functools

import jax
import jax.numpy as jnp
from jax import lax
from jax.experimental import pallas as pl
from jax.experimental.pallas import tpu as pltpu
from jax.experimental.pallas import tpu_sc as plsc

N = 10000
E = 320000
D_IN = 128
H = 256
OUT = 16

NC = 2
NS = 16
NQ = 4
Q = H // NQ
NP = 10240
RPS = NP // NS
CHUNK = 80


def _dense_prologue(x, W_enc, b_enc, W1, b1, W_gcn):
    R = 1000

    def body(x_ref, we_ref, be_ref, w1_ref, b1_ref, wg_ref, out_ref):
        h = jnp.dot(x_ref[...], we_ref[...], preferred_element_type=jnp.float32)
        h = h + be_ref[...]
        h = jax.nn.relu(jnp.dot(h, w1_ref[...], preferred_element_type=jnp.float32) + b1_ref[...])
        out_ref[...] = jnp.dot(h, wg_ref[...], preferred_element_type=jnp.float32)

    return pl.pallas_call(
        body,
        grid=(N // R,),
        in_specs=[
            pl.BlockSpec((R, D_IN), lambda i: (i, 0)),
            pl.BlockSpec((D_IN, H), lambda i: (0, 0)),
            pl.BlockSpec((1, H), lambda i: (0, 0)),
            pl.BlockSpec((H, H), lambda i: (0, 0)),
            pl.BlockSpec((1, H), lambda i: (0, 0)),
            pl.BlockSpec((H, H), lambda i: (0, 0)),
        ],
        out_specs=pl.BlockSpec((R, H), lambda i: (i, 0)),
        out_shape=jax.ShapeDtypeStruct((N, H), jnp.float32),
    )(x, W_enc, b_enc.reshape(1, H), W1, b1.reshape(1, H), W_gcn)


def _scale_rows(p0, p1, xw):
    R = 1000
    nb = N // R

    def body(p0_ref, p1_ref, xw_ref, o0_ref, o1_ref, o2_ref, o3_ref):
        deg = p0_ref[:, 0:1] + p1_ref[:, 0:1] + 1.0
        y = lax.rsqrt(deg) * xw_ref[...]
        o0_ref[...] = y[:, 0 * Q:1 * Q]
        o1_ref[...] = y[:, 1 * Q:2 * Q]
        o2_ref[...] = y[:, 2 * Q:3 * Q]
        o3_ref[...] = y[:, 3 * Q:4 * Q]

    qspec = pl.BlockSpec((R, Q), lambda i: (i, 0))
    return pl.pallas_call(
        body,
        grid=(nb,),
        in_specs=[
            pl.BlockSpec((R, 16), lambda i: (i, 0)),
            pl.BlockSpec((R, 16), lambda i: (i, 0)),
            pl.BlockSpec((R, H), lambda i: (i, 0)),
        ],
        out_specs=[qspec, qspec, qspec, qspec],
        out_shape=[jax.ShapeDtypeStruct((N, Q), jnp.float32)] * NQ,
    )(p0, p1, xw)


def _epilogue(S_parts, y, p0, p1, b_gcn, W_cat, b_cat):
    R = 1000
    nb = N // R

    def body(s0_ref, s1_ref, s2_ref, s3_ref, y0_ref, y1_ref, y2_ref, y3_ref,
             p0_ref, p1_ref, bg_ref, wc_ref, bc_ref, out_ref):
        deg = p0_ref[:, 0:1] + p1_ref[:, 0:1] + 1.0
        dinv = lax.rsqrt(deg)
        a = jnp.concatenate(
            [s0_ref[...] + y0_ref[...], s1_ref[...] + y1_ref[...],
             s2_ref[...] + y2_ref[...], s3_ref[...] + y3_ref[...]], axis=1)
        f = jax.nn.relu(dinv * a + bg_ref[...])
        out_ref[...] = jnp.dot(f, wc_ref[...], preferred_element_type=jnp.float32) + bc_ref[...]

    squarter = pl.BlockSpec((R, Q), lambda i: (i, 0))

    def yq(q):
        return pl.BlockSpec((R, Q), lambda i, q=q: (q * nb + i, 0))

    return pl.pallas_call(
        body,
        grid=(nb,),
        in_specs=[
            squarter, squarter, squarter, squarter,
            yq(0), yq(1), yq(2), yq(3),
            pl.BlockSpec((R, 16), lambda i: (i, 0)),
            pl.BlockSpec((R, 16), lambda i: (i, 0)),
            pl.BlockSpec((1, H), lambda i: (0, 0)),
            pl.BlockSpec((H, OUT + 1), lambda i: (0, 0)),
            pl.BlockSpec((1, OUT + 1), lambda i: (0, 0)),
        ],
        out_specs=pl.BlockSpec((R, OUT + 1), lambda i: (i, 0)),
        out_shape=jax.ShapeDtypeStruct((N, OUT + 1), jnp.float32),
    )(*S_parts, y, y, y, y, p0, p1,
      b_gcn.reshape(1, H), W_cat, b_cat.reshape(1, OUT + 1))


def _make_hist():
    mesh = plsc.VectorSubcoreMesh(
        core_axis_name="c", subcore_axis_name="s", num_cores=NC, num_subcores=NS)
    e_per_sub = E // (NC * NS)
    n_chunks = e_per_sub // CHUNK

    @functools.partial(
        pl.kernel,
        mesh=mesh,
        compiler_params=pltpu.CompilerParams(use_tc_tiling_on_sc=False),
        out_type=jax.ShapeDtypeStruct((NC * NP, 16), jnp.float32),
        scratch_types=[
            pltpu.VMEM_SHARED((NP, 16), jnp.float32),
            pltpu.VMEM((RPS, 16), jnp.float32),
            pltpu.VMEM((CHUNK, 16), jnp.float32),
            pltpu.VMEM((CHUNK,), jnp.int32),
        ],
    )
    def hist(col_hbm, ones_hbm, z_hbm, out_hbm, hist_sp, bounce, ones_v, cidx):
        cid = lax.axis_index("c")
        sid = lax.axis_index("s")
        pltpu.sync_copy(z_hbm, bounce)
        pltpu.sync_copy(bounce, hist_sp.at[pl.ds(sid * RPS, RPS)])
        pltpu.sync_copy(ones_hbm, ones_v)
        plsc.subcore_barrier()

        base = (cid * NS + sid) * e_per_sub

        def step(j, carry):
            pltpu.sync_copy(col_hbm.at[pl.ds(base + j * CHUNK, CHUNK)], cidx)
            pltpu.sync_copy(ones_v, hist_sp.at[cidx], add=True)
            return carry

        lax.fori_loop(0, n_chunks, step, 0)
        plsc.subcore_barrier()
        pltpu.sync_copy(hist_sp.at[pl.ds(sid * RPS, RPS)], bounce)
        pltpu.sync_copy(bounce, out_hbm.at[pl.ds(cid * NP + sid * RPS, RPS)])

    return hist


def _make_scatter():
    mesh = plsc.VectorSubcoreMesh(
        core_axis_name="c", subcore_axis_name="s", num_cores=NC, num_subcores=NS)
    e_per_sub = E // NS
    n_chunks = e_per_sub // CHUNK

    @functools.partial(
        pl.kernel,
        mesh=mesh,
        compiler_params=pltpu.CompilerParams(use_tc_tiling_on_sc=False),
        out_type=jax.ShapeDtypeStruct((NQ * NP, Q), jnp.float32),
        scratch_types=[
            pltpu.VMEM_SHARED((NP, Q), jnp.float32),
            pltpu.VMEM((RPS, Q), jnp.float32),
            pltpu.VMEM((CHUNK, Q), jnp.float32),
            pltpu.VMEM((CHUNK,), jnp.int32),
            pltpu.VMEM((CHUNK,), jnp.int32),
            pltpu.VMEM((CHUNK,), jnp.int32),
            pltpu.SemaphoreType.DMA,
        ],
    )
    def scatter(row_hbm, col_hbm, y_hbm, z_hbm, out_hbm,
                agg_sp, bounce, rows_v, ridx, ridx2, cidx, sem):
        cid = lax.axis_index("c")
        sid = lax.axis_index("s")
        base = sid * e_per_sub
        pltpu.sync_copy(z_hbm, bounce)

        for ph in range(NQ // NC):
            qid = cid * (NQ // NC) + ph
            off = qid * N
            pltpu.sync_copy(bounce, agg_sp.at[pl.ds(sid * RPS, RPS)])
            plsc.subcore_barrier()

            def step(j, carry):
                pltpu.sync_copy(row_hbm.at[pl.ds(base + j * CHUNK, CHUNK)], ridx)
                for i in range(CHUNK // 16):
                    ridx2[pl.ds(i * 16, 16)] = ridx[pl.ds(i * 16, 16)] + off
                pltpu.async_copy(y_hbm.at[ridx2], rows_v, sem).wait()
                pltpu.sync_copy(col_hbm.at[pl.ds(base + j * CHUNK, CHUNK)], cidx)
                pltpu.sync_copy(rows_v, agg_sp.at[cidx], add=True)
                return carry

            lax.fori_loop(0, n_chunks, step, 0)
            plsc.subcore_barrier()
            pltpu.sync_copy(agg_sp.at[pl.ds(sid * RPS, RPS)], bounce)
            pltpu.sync_copy(bounce, out_hbm.at[pl.ds(qid * NP + sid * RPS, RPS)])
            plsc.subcore_barrier()
            pltpu.sync_copy(z_hbm, bounce)

    return scatter


_hist_kernel = _make_hist()
_scatter_kernel = _make_scatter()


def kernel(x, edge_index, W_enc, b_enc, W1, b1, W_gcn, b_gcn, W_out, b_out, W_v, b_v):
    row = edge_index[0]
    col = edge_index[1]
    ones16 = jnp.ones((CHUNK, 16), jnp.float32)
    z16 = jnp.zeros((RPS, 16), jnp.float32)
    zq = jnp.zeros((RPS, Q), jnp.float32)
    W_cat = jnp.concatenate([W_out, W_v], axis=1)
    b_cat = jnp.concatenate([b_out, b_v], axis=0)

    xw = _dense_prologue(x, W_enc, b_enc, W1, b1, W_gcn)
    p = _hist_kernel(col, ones16, z16)
    p0 = p[:N]
    p1 = p[NP:NP + N]
    y = jnp.concatenate(_scale_rows(p0, p1, xw), axis=0)
    S = _scatter_kernel(row, col, y, zq)
    S_parts = [S[q * NP:q * NP + N] for q in range(NQ)]
    return _epilogue(S_parts, y, p0, p1, b_gcn, W_cat, b_cat)


# speedup vs baseline: 17.0691x; 2.3068x over previous
import functools

import jax
import jax.numpy as jnp
from jax import lax
from jax.experimental import pallas as pl
from jax.experimental.pallas import tpu as pltpu
from jax.experimental.pallas import tpu_sc as plsc

N = 10000
E = 320000
D_IN = 128
H = 256
OUT = 16

NC = 2
NS = 16
NQ = 4
Q = H // NQ
NP = 10240
RPS = NP // NS
CHUNK = 80


def _dense_prologue(x, W_enc, b_enc, W1, b1, W_gcn):
    R = 1000

    def body(x_ref, we_ref, be_ref, w1_ref, b1_ref, wg_ref, out_ref):
        h = jnp.dot(x_ref[...], we_ref[...], preferred_element_type=jnp.float32)
        h = h + be_ref[...]
        h = jax.nn.relu(jnp.dot(h, w1_ref[...], preferred_element_type=jnp.float32) + b1_ref[...])
        out_ref[...] = jnp.dot(h, wg_ref[...], preferred_element_type=jnp.float32)

    return pl.pallas_call(
        body,
        grid=(N // R,),
        in_specs=[
            pl.BlockSpec((R, D_IN), lambda i: (i, 0)),
            pl.BlockSpec((D_IN, H), lambda i: (0, 0)),
            pl.BlockSpec((1, H), lambda i: (0, 0)),
            pl.BlockSpec((H, H), lambda i: (0, 0)),
            pl.BlockSpec((1, H), lambda i: (0, 0)),
            pl.BlockSpec((H, H), lambda i: (0, 0)),
        ],
        out_specs=pl.BlockSpec((R, H), lambda i: (i, 0)),
        out_shape=jax.ShapeDtypeStruct((N, H), jnp.float32),
    )(x, W_enc, b_enc.reshape(1, H), W1, b1.reshape(1, H), W_gcn)


def _scale_rows(p0, p1, xw):
    R = 1000
    nb = N // R

    def body(p0_ref, p1_ref, xw_ref, o0_ref, o1_ref, o2_ref, o3_ref):
        deg = p0_ref[:, 0:1] + p1_ref[:, 0:1] + 1.0
        y = lax.rsqrt(deg) * xw_ref[...]
        o0_ref[...] = y[:, 0 * Q:1 * Q]
        o1_ref[...] = y[:, 1 * Q:2 * Q]
        o2_ref[...] = y[:, 2 * Q:3 * Q]
        o3_ref[...] = y[:, 3 * Q:4 * Q]

    qspec = pl.BlockSpec((R, Q), lambda i: (i, 0))
    return pl.pallas_call(
        body,
        grid=(nb,),
        in_specs=[
            pl.BlockSpec((R, 16), lambda i: (i, 0)),
            pl.BlockSpec((R, 16), lambda i: (i, 0)),
            pl.BlockSpec((R, H), lambda i: (i, 0)),
        ],
        out_specs=[qspec, qspec, qspec, qspec],
        out_shape=[jax.ShapeDtypeStruct((N, Q), jnp.float32)] * NQ,
    )(p0, p1, xw)


def _epilogue(S_parts, y, p0, p1, b_gcn, W_cat, b_cat):
    R = 1000
    nb = N // R

    def body(s0_ref, s1_ref, s2_ref, s3_ref, y0_ref, y1_ref, y2_ref, y3_ref,
             p0_ref, p1_ref, bg_ref, wc_ref, bc_ref, out_ref):
        deg = p0_ref[:, 0:1] + p1_ref[:, 0:1] + 1.0
        dinv = lax.rsqrt(deg)
        a = jnp.concatenate(
            [s0_ref[...] + y0_ref[...], s1_ref[...] + y1_ref[...],
             s2_ref[...] + y2_ref[...], s3_ref[...] + y3_ref[...]], axis=1)
        f = jax.nn.relu(dinv * a + bg_ref[...])
        out_ref[...] = jnp.dot(f, wc_ref[...], preferred_element_type=jnp.float32) + bc_ref[...]

    squarter = pl.BlockSpec((R, Q), lambda i: (i, 0))

    def yq(q):
        return pl.BlockSpec((R, Q), lambda i, q=q: (q * nb + i, 0))

    return pl.pallas_call(
        body,
        grid=(nb,),
        in_specs=[
            squarter, squarter, squarter, squarter,
            yq(0), yq(1), yq(2), yq(3),
            pl.BlockSpec((R, 16), lambda i: (i, 0)),
            pl.BlockSpec((R, 16), lambda i: (i, 0)),
            pl.BlockSpec((1, H), lambda i: (0, 0)),
            pl.BlockSpec((H, OUT + 1), lambda i: (0, 0)),
            pl.BlockSpec((1, OUT + 1), lambda i: (0, 0)),
        ],
        out_specs=pl.BlockSpec((R, OUT + 1), lambda i: (i, 0)),
        out_shape=jax.ShapeDtypeStruct((N, OUT + 1), jnp.float32),
    )(*S_parts, y, y, y, y, p0, p1,
      b_gcn.reshape(1, H), W_cat, b_cat.reshape(1, OUT + 1))


def _make_hist():
    mesh = plsc.VectorSubcoreMesh(
        core_axis_name="c", subcore_axis_name="s", num_cores=NC, num_subcores=NS)
    e_per_sub = E // (NC * NS)
    n_chunks = e_per_sub // CHUNK

    @functools.partial(
        pl.kernel,
        mesh=mesh,
        compiler_params=pltpu.CompilerParams(use_tc_tiling_on_sc=False),
        out_type=jax.ShapeDtypeStruct((NC * NP, 16), jnp.float32),
        scratch_types=[
            pltpu.VMEM_SHARED((NP, 16), jnp.float32),
            pltpu.VMEM((RPS, 16), jnp.float32),
            pltpu.VMEM((CHUNK, 16), jnp.float32),
            pltpu.VMEM((CHUNK,), jnp.int32),
        ],
    )
    def hist(col_hbm, ones_hbm, z_hbm, out_hbm, hist_sp, bounce, ones_v, cidx):
        cid = lax.axis_index("c")
        sid = lax.axis_index("s")
        pltpu.sync_copy(z_hbm, bounce)
        pltpu.sync_copy(bounce, hist_sp.at[pl.ds(sid * RPS, RPS)])
        pltpu.sync_copy(ones_hbm, ones_v)
        plsc.subcore_barrier()

        base = (cid * NS + sid) * e_per_sub

        def step(j, carry):
            pltpu.sync_copy(col_hbm.at[pl.ds(base + j * CHUNK, CHUNK)], cidx)
            pltpu.sync_copy(ones_v, hist_sp.at[cidx], add=True)
            return carry

        lax.fori_loop(0, n_chunks, step, 0)
        plsc.subcore_barrier()
        pltpu.sync_copy(hist_sp.at[pl.ds(sid * RPS, RPS)], bounce)
        pltpu.sync_copy(bounce, out_hbm.at[pl.ds(cid * NP + sid * RPS, RPS)])

    return hist


def _make_scatter():
    mesh = plsc.VectorSubcoreMesh(
        core_axis_name="c", subcore_axis_name="s", num_cores=NC, num_subcores=NS)
    e_per_sub = E // NS
    n_chunks = e_per_sub // CHUNK
    ZB = 128
    NZ = RPS // ZB

    @functools.partial(
        pl.kernel,
        mesh=mesh,
        compiler_params=pltpu.CompilerParams(use_tc_tiling_on_sc=False),
        out_type=jax.ShapeDtypeStruct((NQ * NP, Q), jnp.float32),
        scratch_types=[
            pltpu.VMEM_SHARED((NP, Q), jnp.float32),
            pltpu.VMEM((ZB, Q), jnp.float32),
            pltpu.VMEM((CHUNK, Q), jnp.float32),
            pltpu.VMEM((CHUNK, Q), jnp.float32),
            pltpu.VMEM((n_chunks, CHUNK), jnp.int32),
            pltpu.VMEM((n_chunks, CHUNK), jnp.int32),
            pltpu.SemaphoreType.DMA,
            pltpu.SemaphoreType.DMA,
        ],
    )
    def scatter(row2d_hbm, col2d_hbm, y_hbm, z_hbm, out_hbm,
                agg_sp, bounce, rows0, rows1, ridx_v, cidx_v, sem0, sem1):
        cid = lax.axis_index("c")
        sid = lax.axis_index("s")
        rows = (rows0, rows1)
        sems = (sem0, sem1)
        pltpu.sync_copy(col2d_hbm.at[pl.ds(sid * n_chunks, n_chunks)], cidx_v)

        for ph in range(NQ // NC):
            qid = cid * (NQ // NC) + ph
            off = qid * N
            pltpu.sync_copy(row2d_hbm.at[pl.ds(sid * n_chunks, n_chunks)], ridx_v)

            def shift(j, carry):
                for i in range(CHUNK // 16):
                    ridx_v[j, pl.ds(i * 16, 16)] = ridx_v[j, pl.ds(i * 16, 16)] + off
                return carry

            lax.fori_loop(0, n_chunks, shift, 0)

            pltpu.sync_copy(z_hbm, bounce)
            for k in range(NZ):
                pltpu.sync_copy(bounce, agg_sp.at[pl.ds(sid * RPS + k * ZB, ZB)])
            plsc.subcore_barrier()

            for b in range(2):
                pltpu.async_copy(y_hbm.at[ridx_v.at[b]], rows[b], sems[b])

            def step(j2, carry):
                for b in range(2):
                    j = j2 * 2 + b
                    pltpu.make_async_copy(y_hbm.at[ridx_v.at[j]], rows[b], sems[b]).wait()
                    pltpu.sync_copy(rows[b], agg_sp.at[cidx_v.at[j]], add=True)

                    @pl.when(j + 2 < n_chunks)
                    def _():
                        pltpu.async_copy(y_hbm.at[ridx_v.at[j + 2]], rows[b], sems[b])
                return carry

            lax.fori_loop(0, n_chunks // 2, step, 0)
            plsc.subcore_barrier()
            for k in range(NZ):
                pltpu.sync_copy(agg_sp.at[pl.ds(sid * RPS + k * ZB, ZB)], bounce)
                pltpu.sync_copy(
                    bounce, out_hbm.at[pl.ds(qid * NP + sid * RPS + k * ZB, ZB)])

    return scatter


_hist_kernel = _make_hist()
_scatter_kernel = _make_scatter()


def kernel(x, edge_index, W_enc, b_enc, W1, b1, W_gcn, b_gcn, W_out, b_out, W_v, b_v):
    row2d = edge_index[0].reshape(E // CHUNK, CHUNK)
    col = edge_index[1]
    col2d = col.reshape(E // CHUNK, CHUNK)
    ones16 = jnp.ones((CHUNK, 16), jnp.float32)
    z16 = jnp.zeros((RPS, 16), jnp.float32)
    zq = jnp.zeros((128, Q), jnp.float32)
    W_cat = jnp.concatenate([W_out, W_v], axis=1)
    b_cat = jnp.concatenate([b_out, b_v], axis=0)

    xw = _dense_prologue(x, W_enc, b_enc, W1, b1, W_gcn)
    p = _hist_kernel(col, ones16, z16)
    p0 = p[:N]
    p1 = p[NP:NP + N]
    y = jnp.concatenate(_scale_rows(p0, p1, xw), axis=0)
    S = _scatter_kernel(row2d, col2d, y, zq)
    S_parts = [S[q * NP:q * NP + N] for q in range(NQ)]
    return _epilogue(S_parts, y, p0, p1, b_gcn, W_cat, b_cat)


# speedup vs baseline: 25.5951x; 1.4995x over previous
import functools

import jax
import jax.numpy as jnp
from jax import lax
from jax.experimental import pallas as pl
from jax.experimental.pallas import tpu as pltpu
from jax.experimental.pallas import tpu_sc as plsc

N = 10000
E = 320000
D_IN = 128
H = 256
OUT = 16

NC = 2
NS = 16
NQ = 4
Q = H // NQ
NP = 10240
RPS = NP // NS
CHUNK = 80


def _dense_prologue(x, W_enc, b_enc, W1, b1, W_gcn):
    R = 1000

    def body(x_ref, we_ref, be_ref, w1_ref, b1_ref, wg_ref, out_ref):
        h = jnp.dot(x_ref[...], we_ref[...], preferred_element_type=jnp.float32)
        h = h + be_ref[...]
        h = jax.nn.relu(jnp.dot(h, w1_ref[...], preferred_element_type=jnp.float32) + b1_ref[...])
        out_ref[...] = jnp.dot(h, wg_ref[...], preferred_element_type=jnp.float32)

    return pl.pallas_call(
        body,
        grid=(N // R,),
        in_specs=[
            pl.BlockSpec((R, D_IN), lambda i: (i, 0)),
            pl.BlockSpec((D_IN, H), lambda i: (0, 0)),
            pl.BlockSpec((1, H), lambda i: (0, 0)),
            pl.BlockSpec((H, H), lambda i: (0, 0)),
            pl.BlockSpec((1, H), lambda i: (0, 0)),
            pl.BlockSpec((H, H), lambda i: (0, 0)),
        ],
        out_specs=pl.BlockSpec((R, H), lambda i: (i, 0)),
        out_shape=jax.ShapeDtypeStruct((N, H), jnp.float32),
    )(x, W_enc, b_enc.reshape(1, H), W1, b1.reshape(1, H), W_gcn)


def _scale_rows(p0, p1, xw):
    R = 1000
    nb = N // R

    def body(p0_ref, p1_ref, xw_ref, o0_ref, o1_ref, o2_ref, o3_ref):
        deg = p0_ref[:, 0:1] + p1_ref[:, 0:1] + 1.0
        y = lax.rsqrt(deg) * xw_ref[...]
        o0_ref[...] = y[:, 0 * Q:1 * Q]
        o1_ref[...] = y[:, 1 * Q:2 * Q]
        o2_ref[...] = y[:, 2 * Q:3 * Q]
        o3_ref[...] = y[:, 3 * Q:4 * Q]

    qspec = pl.BlockSpec((R, Q), lambda i: (i, 0))
    return pl.pallas_call(
        body,
        grid=(nb,),
        in_specs=[
            pl.BlockSpec((R, 16), lambda i: (i, 0)),
            pl.BlockSpec((R, 16), lambda i: (i, 0)),
            pl.BlockSpec((R, H), lambda i: (i, 0)),
        ],
        out_specs=[qspec, qspec, qspec, qspec],
        out_shape=[jax.ShapeDtypeStruct((N, Q), jnp.float32)] * NQ,
    )(p0, p1, xw)


def _epilogue(S_parts, y, p0, p1, b_gcn, W_cat, b_cat):
    R = 1000
    nb = N // R

    def body(s0_ref, s1_ref, s2_ref, s3_ref, y0_ref, y1_ref, y2_ref, y3_ref,
             p0_ref, p1_ref, bg_ref, wc_ref, bc_ref, out_ref):
        deg = p0_ref[:, 0:1] + p1_ref[:, 0:1] + 1.0
        dinv = lax.rsqrt(deg)
        a = jnp.concatenate(
            [s0_ref[...] + y0_ref[...], s1_ref[...] + y1_ref[...],
             s2_ref[...] + y2_ref[...], s3_ref[...] + y3_ref[...]], axis=1)
        f = jax.nn.relu(dinv * a + bg_ref[...])
        out_ref[...] = jnp.dot(f, wc_ref[...], preferred_element_type=jnp.float32) + bc_ref[...]

    squarter = pl.BlockSpec((R, Q), lambda i: (i, 0))

    def yq(q):
        return pl.BlockSpec((R, Q), lambda i, q=q: (q * nb + i, 0))

    return pl.pallas_call(
        body,
        grid=(nb,),
        in_specs=[
            squarter, squarter, squarter, squarter,
            yq(0), yq(1), yq(2), yq(3),
            pl.BlockSpec((R, 16), lambda i: (i, 0)),
            pl.BlockSpec((R, 16), lambda i: (i, 0)),
            pl.BlockSpec((1, H), lambda i: (0, 0)),
            pl.BlockSpec((H, OUT + 1), lambda i: (0, 0)),
            pl.BlockSpec((1, OUT + 1), lambda i: (0, 0)),
        ],
        out_specs=pl.BlockSpec((R, OUT + 1), lambda i: (i, 0)),
        out_shape=jax.ShapeDtypeStruct((N, OUT + 1), jnp.float32),
    )(*S_parts, y, y, y, y, p0, p1,
      b_gcn.reshape(1, H), W_cat, b_cat.reshape(1, OUT + 1))


def _make_hist():
    mesh = plsc.VectorSubcoreMesh(
        core_axis_name="c", subcore_axis_name="s", num_cores=NC, num_subcores=NS)
    e_per_sub = E // (NC * NS)
    n_chunks = e_per_sub // CHUNK

    @functools.partial(
        pl.kernel,
        mesh=mesh,
        compiler_params=pltpu.CompilerParams(use_tc_tiling_on_sc=False),
        out_type=jax.ShapeDtypeStruct((NC * NP, 16), jnp.float32),
        scratch_types=[
            pltpu.VMEM_SHARED((NP, 16), jnp.float32),
            pltpu.VMEM((RPS, 16), jnp.float32),
            pltpu.VMEM((CHUNK, 16), jnp.float32),
            pltpu.VMEM((n_chunks, CHUNK), jnp.int32),
            pltpu.SemaphoreType.DMA,
        ],
    )
    def hist(col2d_hbm, ones_hbm, z_hbm, out_hbm, hist_sp, bounce, ones_v,
             cidx_v, sem):
        cid = lax.axis_index("c")
        sid = lax.axis_index("s")
        wid = cid * NS + sid
        pltpu.sync_copy(col2d_hbm.at[pl.ds(wid * n_chunks, n_chunks)], cidx_v)
        pltpu.sync_copy(ones_hbm, ones_v)
        pltpu.sync_copy(z_hbm, bounce)
        pltpu.sync_copy(bounce, hist_sp.at[pl.ds(sid * RPS, RPS)])
        plsc.subcore_barrier()

        def step(j2, carry):
            for k in range(5):
                pltpu.async_copy(
                    ones_v, hist_sp.at[cidx_v.at[j2 * 5 + k]], sem, add=True)
            for k in range(5):
                pltpu.make_async_copy(
                    ones_v, hist_sp.at[cidx_v.at[j2 * 5 + k]], sem).wait()
            return carry

        lax.fori_loop(0, n_chunks // 5, step, 0)
        plsc.subcore_barrier()
        pltpu.sync_copy(hist_sp.at[pl.ds(sid * RPS, RPS)], bounce)
        pltpu.sync_copy(bounce, out_hbm.at[pl.ds(cid * NP + sid * RPS, RPS)])

    return hist


def _make_scatter():
    mesh = plsc.VectorSubcoreMesh(
        core_axis_name="c", subcore_axis_name="s", num_cores=NC, num_subcores=NS)
    e_per_sub = E // NS
    n_chunks = e_per_sub // CHUNK
    ZB = 128
    NZ = RPS // ZB

    @functools.partial(
        pl.kernel,
        mesh=mesh,
        compiler_params=pltpu.CompilerParams(use_tc_tiling_on_sc=False),
        out_type=jax.ShapeDtypeStruct((NQ * NP, Q), jnp.float32),
        scratch_types=[
            pltpu.VMEM_SHARED((NP, Q), jnp.float32),
            pltpu.VMEM((ZB, Q), jnp.float32),
            pltpu.VMEM((CHUNK, Q), jnp.float32),
            pltpu.VMEM((CHUNK, Q), jnp.float32),
            pltpu.VMEM((CHUNK, Q), jnp.float32),
            pltpu.VMEM((CHUNK, Q), jnp.float32),
            pltpu.VMEM((CHUNK, Q), jnp.float32),
            pltpu.VMEM((n_chunks, CHUNK), jnp.int32),
            pltpu.VMEM((n_chunks, CHUNK), jnp.int32),
            pltpu.SemaphoreType.DMA,
            pltpu.SemaphoreType.DMA,
            pltpu.SemaphoreType.DMA,
            pltpu.SemaphoreType.DMA,
            pltpu.SemaphoreType.DMA,
            pltpu.SemaphoreType.DMA,
            pltpu.SemaphoreType.DMA,
            pltpu.SemaphoreType.DMA,
            pltpu.SemaphoreType.DMA,
            pltpu.SemaphoreType.DMA,
        ],
    )
    def scatter(row2d_hbm, col2d_hbm, y_hbm, z_hbm, out_hbm,
                agg_sp, bounce, rows0, rows1, rows2, rows3, rows4,
                ridx_v, cidx_v,
                gs0, gs1, gs2, gs3, gs4, ss0, ss1, ss2, ss3, ss4):
        cid = lax.axis_index("c")
        sid = lax.axis_index("s")
        rows = (rows0, rows1, rows2, rows3, rows4)
        gs = (gs0, gs1, gs2, gs3, gs4)
        ss = (ss0, ss1, ss2, ss3, ss4)
        pltpu.sync_copy(col2d_hbm.at[pl.ds(sid * n_chunks, n_chunks)], cidx_v)

        for ph in range(NQ // NC):
            qid = cid * (NQ // NC) + ph
            off = qid * N
            pltpu.sync_copy(row2d_hbm.at[pl.ds(sid * n_chunks, n_chunks)], ridx_v)

            def shift(j, carry):
                for i in range(CHUNK // 16):
                    ridx_v[j, pl.ds(i * 16, 16)] = ridx_v[j, pl.ds(i * 16, 16)] + off
                return carry

            lax.fori_loop(0, n_chunks, shift, 0)

            pltpu.sync_copy(z_hbm, bounce)
            for k in range(NZ):
                pltpu.sync_copy(bounce, agg_sp.at[pl.ds(sid * RPS + k * ZB, ZB)])
            plsc.subcore_barrier()

            for b in range(5):
                pltpu.async_copy(y_hbm.at[ridx_v.at[b]], rows[b], gs[b])

            def step(j2, carry):
                for b in range(5):
                    j = j2 * 5 + b
                    bp = (b - 1) % 5
                    jp = j - 1
                    pltpu.make_async_copy(
                        y_hbm.at[ridx_v.at[j]], rows[b], gs[b]).wait()
                    pltpu.async_copy(
                        rows[b], agg_sp.at[cidx_v.at[j]], ss[b], add=True)

                    @pl.when((jp >= 0) & (jp + 5 < n_chunks))
                    def _():
                        pltpu.make_async_copy(
                            rows[bp], agg_sp.at[cidx_v.at[jp]], ss[bp]).wait()
                        pltpu.async_copy(
                            y_hbm.at[ridx_v.at[jp + 5]], rows[bp], gs[bp])
                return carry

            lax.fori_loop(0, n_chunks // 5, step, 0)
            for j in range(n_chunks - 5, n_chunks):
                pltpu.make_async_copy(
                    rows[j % 5], agg_sp.at[cidx_v.at[j]], ss[j % 5]).wait()
            plsc.subcore_barrier()
            for k in range(NZ):
                pltpu.sync_copy(agg_sp.at[pl.ds(sid * RPS + k * ZB, ZB)], bounce)
                pltpu.sync_copy(
                    bounce, out_hbm.at[pl.ds(qid * NP + sid * RPS + k * ZB, ZB)])

    return scatter


_hist_kernel = _make_hist()
_scatter_kernel = _make_scatter()


def kernel(x, edge_index, W_enc, b_enc, W1, b1, W_gcn, b_gcn, W_out, b_out, W_v, b_v):
    row2d = edge_index[0].reshape(E // CHUNK, CHUNK)
    col2d = edge_index[1].reshape(E // CHUNK, CHUNK)
    ones16 = jnp.ones((CHUNK, 16), jnp.float32)
    z16 = jnp.zeros((RPS, 16), jnp.float32)
    zq = jnp.zeros((128, Q), jnp.float32)
    W_cat = jnp.concatenate([W_out, W_v], axis=1)
    b_cat = jnp.concatenate([b_out, b_v], axis=0)

    xw = _dense_prologue(x, W_enc, b_enc, W1, b1, W_gcn)
    p = _hist_kernel(col2d, ones16, z16)
    p0 = p[:N]
    p1 = p[NP:NP + N]
    y = jnp.concatenate(_scale_rows(p0, p1, xw), axis=0)
    S = _scatter_kernel(row2d, col2d, y, zq)
    S_parts = [S[q * NP:q * NP + N] for q in range(NQ)]
    return _epilogue(S_parts, y, p0, p1, b_gcn, W_cat, b_cat)


# speedup vs baseline: 27.1583x; 1.0611x over previous
import functools

import jax
import jax.numpy as jnp
from jax import lax
from jax.experimental import pallas as pl
from jax.experimental.pallas import tpu as pltpu
from jax.experimental.pallas import tpu_sc as plsc

N = 10000
E = 320000
D_IN = 128
H = 256
OUT = 16

NC = 2
NS = 16
NQ = 4
Q = H // NQ
NP = 10240
RPS = NP // NS
CHUNK = 80


def _dense_prologue(x, W_enc, b_enc, W1, b1, W_gcn):
    R = 1000

    def body(x_ref, we_ref, be_ref, w1_ref, b1_ref, wg_ref, out_ref):
        h = jnp.dot(x_ref[...], we_ref[...], preferred_element_type=jnp.float32)
        h = h + be_ref[...]
        h = jax.nn.relu(jnp.dot(h, w1_ref[...], preferred_element_type=jnp.float32) + b1_ref[...])
        out_ref[...] = jnp.dot(h, wg_ref[...], preferred_element_type=jnp.float32)

    return pl.pallas_call(
        body,
        grid=(N // R,),
        in_specs=[
            pl.BlockSpec((R, D_IN), lambda i: (i, 0)),
            pl.BlockSpec((D_IN, H), lambda i: (0, 0)),
            pl.BlockSpec((1, H), lambda i: (0, 0)),
            pl.BlockSpec((H, H), lambda i: (0, 0)),
            pl.BlockSpec((1, H), lambda i: (0, 0)),
            pl.BlockSpec((H, H), lambda i: (0, 0)),
        ],
        out_specs=pl.BlockSpec((R, H), lambda i: (i, 0)),
        out_shape=jax.ShapeDtypeStruct((N, H), jnp.float32),
    )(x, W_enc, b_enc.reshape(1, H), W1, b1.reshape(1, H), W_gcn)


def _scale_rows(p, xw):
    R = 1000
    nb = N // R

    def body(p0_ref, p1_ref, xw_ref, out_ref):
        deg = p0_ref[0, :, 0:1] + p1_ref[0, :, 0:1] + 1.0
        y = lax.rsqrt(deg) * xw_ref[...]
        for q in range(NQ):
            out_ref[q] = y[:, q * Q:(q + 1) * Q]

    return pl.pallas_call(
        body,
        grid=(nb,),
        in_specs=[
            pl.BlockSpec((1, R, 16), lambda i: (0, i, 0)),
            pl.BlockSpec((1, R, 16), lambda i: (1, i, 0)),
            pl.BlockSpec((R, H), lambda i: (i, 0)),
        ],
        out_specs=pl.BlockSpec((NQ, R, Q), lambda i: (0, i, 0)),
        out_shape=jax.ShapeDtypeStruct((NQ, N, Q), jnp.float32),
    )(p, p, xw)


def _epilogue(S, y, p, b_gcn, W_cat, b_cat):
    R = 1000
    nb = N // R

    def body(s_ref, y_ref, p0_ref, p1_ref, bg_ref, wc_ref, bc_ref, out_ref):
        deg = p0_ref[0, :, 0:1] + p1_ref[0, :, 0:1] + 1.0
        dinv = lax.rsqrt(deg)
        a = jnp.concatenate(
            [s_ref[q] + y_ref[q] for q in range(NQ)], axis=1)
        f = jax.nn.relu(dinv * a + bg_ref[...])
        out_ref[...] = jnp.dot(f, wc_ref[...], preferred_element_type=jnp.float32) + bc_ref[...]

    return pl.pallas_call(
        body,
        grid=(nb,),
        in_specs=[
            pl.BlockSpec((NQ, R, Q), lambda i: (0, i, 0)),
            pl.BlockSpec((NQ, R, Q), lambda i: (0, i, 0)),
            pl.BlockSpec((1, R, 16), lambda i: (0, i, 0)),
            pl.BlockSpec((1, R, 16), lambda i: (1, i, 0)),
            pl.BlockSpec((1, H), lambda i: (0, 0)),
            pl.BlockSpec((H, OUT + 1), lambda i: (0, 0)),
            pl.BlockSpec((1, OUT + 1), lambda i: (0, 0)),
        ],
        out_specs=pl.BlockSpec((R, OUT + 1), lambda i: (i, 0)),
        out_shape=jax.ShapeDtypeStruct((N, OUT + 1), jnp.float32),
    )(S, y, p, p,
      b_gcn.reshape(1, H), W_cat, b_cat.reshape(1, OUT + 1))


def _make_hist():
    mesh = plsc.VectorSubcoreMesh(
        core_axis_name="c", subcore_axis_name="s", num_cores=NC, num_subcores=NS)
    e_per_sub = E // (NC * NS)
    n_chunks = e_per_sub // CHUNK

    @functools.partial(
        pl.kernel,
        mesh=mesh,
        compiler_params=pltpu.CompilerParams(use_tc_tiling_on_sc=False),
        out_type=jax.ShapeDtypeStruct((NC, NP, 16), jnp.float32),
        scratch_types=[
            pltpu.VMEM_SHARED((NP, 16), jnp.float32),
            pltpu.VMEM((RPS, 16), jnp.float32),
            pltpu.VMEM((CHUNK, 16), jnp.float32),
            pltpu.VMEM((n_chunks, CHUNK), jnp.int32),
            pltpu.SemaphoreType.DMA,
        ],
    )
    def hist(col2d_hbm, ones_hbm, z_hbm, out_hbm, hist_sp, bounce, ones_v,
             cidx_v, sem):
        cid = lax.axis_index("c")
        sid = lax.axis_index("s")
        wid = cid * NS + sid
        pltpu.sync_copy(col2d_hbm.at[pl.ds(wid * n_chunks, n_chunks)], cidx_v)
        pltpu.sync_copy(ones_hbm, ones_v)
        pltpu.sync_copy(z_hbm, bounce)
        pltpu.sync_copy(bounce, hist_sp.at[pl.ds(sid * RPS, RPS)])
        plsc.subcore_barrier()

        def step(j2, carry):
            for k in range(5):
                pltpu.async_copy(
                    ones_v, hist_sp.at[cidx_v.at[j2 * 5 + k]], sem, add=True)
            for k in range(5):
                pltpu.make_async_copy(
                    ones_v, hist_sp.at[cidx_v.at[j2 * 5 + k]], sem).wait()
            return carry

        lax.fori_loop(0, n_chunks // 5, step, 0)
        plsc.subcore_barrier()
        pltpu.sync_copy(hist_sp.at[pl.ds(sid * RPS, RPS)], bounce)
        pltpu.sync_copy(bounce, out_hbm.at[cid, pl.ds(sid * RPS, RPS)])

    return hist


def _make_scatter():
    mesh = plsc.VectorSubcoreMesh(
        core_axis_name="c", subcore_axis_name="s", num_cores=NC, num_subcores=NS)
    e_per_sub = E // NS
    n_chunks = e_per_sub // CHUNK
    ZB = 128
    NZ = RPS // ZB

    @functools.partial(
        pl.kernel,
        mesh=mesh,
        compiler_params=pltpu.CompilerParams(use_tc_tiling_on_sc=False),
        out_type=jax.ShapeDtypeStruct((NQ, NP, Q), jnp.float32),
        scratch_types=[
            pltpu.VMEM_SHARED((NP, Q), jnp.float32),
            pltpu.VMEM((ZB, Q), jnp.float32),
            pltpu.VMEM((CHUNK, Q), jnp.float32),
            pltpu.VMEM((CHUNK, Q), jnp.float32),
            pltpu.VMEM((CHUNK, Q), jnp.float32),
            pltpu.VMEM((CHUNK, Q), jnp.float32),
            pltpu.VMEM((CHUNK, Q), jnp.float32),
            pltpu.VMEM((n_chunks, CHUNK), jnp.int32),
            pltpu.VMEM((n_chunks, CHUNK), jnp.int32),
            pltpu.SemaphoreType.DMA,
            pltpu.SemaphoreType.DMA,
            pltpu.SemaphoreType.DMA,
            pltpu.SemaphoreType.DMA,
            pltpu.SemaphoreType.DMA,
            pltpu.SemaphoreType.DMA,
            pltpu.SemaphoreType.DMA,
            pltpu.SemaphoreType.DMA,
            pltpu.SemaphoreType.DMA,
            pltpu.SemaphoreType.DMA,
        ],
    )
    def scatter(row2d_hbm, col2d_hbm, y_hbm, z_hbm, out_hbm,
                agg_sp, bounce, rows0, rows1, rows2, rows3, rows4,
                ridx_v, cidx_v,
                gs0, gs1, gs2, gs3, gs4, ss0, ss1, ss2, ss3, ss4):
        cid = lax.axis_index("c")
        sid = lax.axis_index("s")
        rows = (rows0, rows1, rows2, rows3, rows4)
        gs = (gs0, gs1, gs2, gs3, gs4)
        ss = (ss0, ss1, ss2, ss3, ss4)
        pltpu.sync_copy(col2d_hbm.at[pl.ds(sid * n_chunks, n_chunks)], cidx_v)

        for ph in range(NQ // NC):
            qid = cid * (NQ // NC) + ph
            off = qid * N
            pltpu.sync_copy(row2d_hbm.at[pl.ds(sid * n_chunks, n_chunks)], ridx_v)

            def shift(j, carry):
                for i in range(CHUNK // 16):
                    ridx_v[j, pl.ds(i * 16, 16)] = ridx_v[j, pl.ds(i * 16, 16)] + off
                return carry

            lax.fori_loop(0, n_chunks, shift, 0)

            pltpu.sync_copy(z_hbm, bounce)
            for k in range(NZ):
                pltpu.sync_copy(bounce, agg_sp.at[pl.ds(sid * RPS + k * ZB, ZB)])
            plsc.subcore_barrier()

            for b in range(5):
                pltpu.async_copy(y_hbm.at[ridx_v.at[b]], rows[b], gs[b])

            def step(j2, carry):
                for b in range(5):
                    j = j2 * 5 + b
                    bp = (b - 1) % 5
                    jp = j - 1
                    pltpu.make_async_copy(
                        y_hbm.at[ridx_v.at[j]], rows[b], gs[b]).wait()
                    pltpu.async_copy(
                        rows[b], agg_sp.at[cidx_v.at[j]], ss[b], add=True)

                    @pl.when((jp >= 0) & (jp + 5 < n_chunks))
                    def _():
                        pltpu.make_async_copy(
                            rows[bp], agg_sp.at[cidx_v.at[jp]], ss[bp]).wait()
                        pltpu.async_copy(
                            y_hbm.at[ridx_v.at[jp + 5]], rows[bp], gs[bp])
                return carry

            lax.fori_loop(0, n_chunks // 5, step, 0)
            for j in range(n_chunks - 5, n_chunks):
                pltpu.make_async_copy(
                    rows[j % 5], agg_sp.at[cidx_v.at[j]], ss[j % 5]).wait()
            plsc.subcore_barrier()
            for k in range(NZ):
                pltpu.sync_copy(agg_sp.at[pl.ds(sid * RPS + k * ZB, ZB)], bounce)
                pltpu.sync_copy(
                    bounce, out_hbm.at[qid, pl.ds(sid * RPS + k * ZB, ZB)])

    return scatter


_hist_kernel = _make_hist()
_scatter_kernel = _make_scatter()


def kernel(x, edge_index, W_enc, b_enc, W1, b1, W_gcn, b_gcn, W_out, b_out, W_v, b_v):
    row2d = edge_index[0].reshape(E // CHUNK, CHUNK)
    col2d = edge_index[1].reshape(E // CHUNK, CHUNK)
    ones16 = jnp.ones((CHUNK, 16), jnp.float32)
    z16 = jnp.zeros((RPS, 16), jnp.float32)
    zq = jnp.zeros((128, Q), jnp.float32)
    W_cat = jnp.concatenate([W_out, W_v], axis=1)
    b_cat = jnp.concatenate([b_out, b_v], axis=0)

    xw = _dense_prologue(x, W_enc, b_enc, W1, b1, W_gcn)
    p = _hist_kernel(col2d, ones16, z16)
    y = _scale_rows(p, xw)
    S = _scatter_kernel(row2d, col2d, y.reshape(NQ * N, Q), zq)
    return _epilogue(S[:, :N], y, p, b_gcn, W_cat, b_cat)
